# baseline (speedup 1.0000x reference)
import functools

import jax
import jax.numpy as jnp
from jax import lax
from jax.experimental import pallas as pl
from jax.experimental.pallas import tpu as pltpu
from jax.experimental.pallas import tpu_sc as plsc

PH, PW = 7, 7
R = 5000
RP = 5024
NB = PH * PW
NE = NB * 4
NEP = 208
C = 256
RB = 1256
OUTR = C * NB

LEVEL_W = (256, 128, 64, 32)
LEVEL_BASE = (0, 65536, 81920, 86016)

NW = 32
RPW = 160
NBLK = RPW // 8


def _tc_index_kernel(rois_ref, idx_ref, w_ref):
    rois = rois_ref[...]
    x1 = rois[:, 1:2]
    y1 = rois[:, 2:3]
    x2 = rois[:, 3:4]
    y2 = rois[:, 4:5]
    bw = x2 - x1 + 1.0
    bh = y2 - y1 + 1.0
    fid = jnp.clip(
        jnp.floor(2.0 + jnp.log2(jnp.sqrt(bw * bh) / 224.0 + 1e-6)), 0.0, 3.0
    ).astype(jnp.int32)
    scale = jnp.where(
        fid == 0, 0.25, jnp.where(fid == 1, 0.125, jnp.where(fid == 2, 0.0625, 0.03125))
    ).astype(jnp.float32)
    wl = jnp.where(fid == 0, LEVEL_W[0],
                   jnp.where(fid == 1, LEVEL_W[1],
                             jnp.where(fid == 2, LEVEL_W[2], LEVEL_W[3])))
    basel = jnp.where(fid == 0, LEVEL_BASE[0],
                      jnp.where(fid == 1, LEVEL_BASE[1],
                                jnp.where(fid == 2, LEVEL_BASE[2], LEVEL_BASE[3])))

    lane = lax.broadcasted_iota(jnp.int32, (RB, NEP), 1)
    live = lane < NE
    k = lane // 4
    corner = lane - 4 * k
    bi = k // PW
    bj = k - PW * bi
    dy = corner // 2
    dx = corner - 2 * dy

    x1s = x1 * scale
    y1s = y1 * scale
    roi_w = jnp.maximum(x2 * scale - x1s, 1.0)
    roi_h = jnp.maximum(y2 * scale - y1s, 1.0)
    bin_w = roi_w / PW
    bin_h = roi_h / PH
    px = x1s + (bj.astype(jnp.float32) + 0.5) * bin_w
    py = y1s + (bi.astype(jnp.float32) + 0.5) * bin_h
    x0f = jnp.floor(px)
    y0f = jnp.floor(py)
    lx = px - x0f
    ly = py - y0f
    hi = wl - 1
    x0 = jnp.clip(x0f.astype(jnp.int32), 0, hi)
    x1i = jnp.clip(x0 + 1, 0, hi)
    y0 = jnp.clip(y0f.astype(jnp.int32), 0, hi)
    y1i = jnp.clip(y0 + 1, 0, hi)
    ys = jnp.where(dy == 0, y0, y1i)
    xs = jnp.where(dx == 0, x0, x1i)
    wy = jnp.where(dy == 0, 1.0 - ly, ly)
    wx = jnp.where(dx == 0, 1.0 - lx, lx)
    idx_ref[...] = jnp.where(live, basel + ys * wl + xs, 0)
    w_ref[...] = jnp.where(live, wy * wx, 0.0)


def _tc_indices(rois, interpret=False):
    return pl.pallas_call(
        _tc_index_kernel,
        grid=(RP // RB,),
        in_specs=[pl.BlockSpec((RB, 5), lambda i: (i, 0))],
        out_specs=[
            pl.BlockSpec((RB, NEP), lambda i: (i, 0)),
            pl.BlockSpec((RB, NEP), lambda i: (i, 0)),
        ],
        out_shape=[
            jax.ShapeDtypeStruct((RP, NEP), jnp.int32),
            jax.ShapeDtypeStruct((RP, NEP), jnp.float32),
        ],
        interpret=interpret,
    )(rois)


@functools.cache
def _sc_gather_fn():
    mesh = plsc.VectorSubcoreMesh(
        core_axis_name="c", subcore_axis_name="s", num_cores=2, num_subcores=16
    )
    return functools.partial(
        pl.kernel,
        out_type=jax.ShapeDtypeStruct((R * OUTR,), jnp.float32),
        mesh=mesh,
        scratch_types=[
            pltpu.VMEM((2 * 8 * NEP,), jnp.int32),
            pltpu.VMEM((2 * 8 * NEP,), jnp.float32),
            pltpu.VMEM((2, NEP, C), jnp.float32),
            pltpu.VMEM((OUTR,), jnp.float32),
            pltpu.SemaphoreType.DMA,
            pltpu.SemaphoreType.DMA,
        ],
        compiler_params=pltpu.CompilerParams(needs_layout_passes=False),
    )(_sc_gather_body)


def _sc_gather_body(idx_hbm, w_hbm, table_hbm, out_hbm, sbidx, sbw, rows_v, acc_v,
                    gsem0, gsem1):
    wid = lax.axis_index("s") * 2 + lax.axis_index("c")
    rstart = 156 * wid + jnp.minimum(wid, 8)
    count = jnp.where(wid < 8, 157, 156)
    lane49 = lax.iota(jnp.int32, 16) * NB

    def stage(blk, sb):
        off = (rstart + 8 * blk) * NEP
        pltpu.sync_copy(idx_hbm.at[pl.ds(off, 8 * NEP)],
                        sbidx.at[pl.ds(sb * 8 * NEP, 8 * NEP)])
        pltpu.sync_copy(w_hbm.at[pl.ds(off, 8 * NEP)],
                        sbw.at[pl.ds(sb * 8 * NEP, 8 * NEP)])

    def fire(sb, eoff, rb, sem):
        so = sb * 8 * NEP + eoff
        pltpu.async_copy(
            table_hbm.at[sbidx.at[pl.ds(so, 128)]],
            rows_v.at[rb, pl.ds(0, 128)], sem)
        pltpu.async_copy(
            table_hbm.at[sbidx.at[pl.ds(so + 128, 80)]],
            rows_v.at[rb, pl.ds(128, 80)], sem)

    def wait(sb, eoff, rb, sem):
        so = sb * 8 * NEP + eoff
        pltpu.make_async_copy(
            table_hbm.at[sbidx.at[pl.ds(so, 128)]],
            rows_v.at[rb, pl.ds(0, 128)], sem).wait()
        pltpu.make_async_copy(
            table_hbm.at[sbidx.at[pl.ds(so + 128, 80)]],
            rows_v.at[rb, pl.ds(128, 80)], sem).wait()

    def fire_roi(rn):
        sbn = (rn // 8) % 2
        eoff = pl.multiple_of(NEP * (rn % 8), 16)

        @pl.when(rn % 2 == 0)
        def _():
            @pl.when(sbn == 0)
            def _():
                fire(0, eoff, 0, gsem0)

            @pl.when(sbn == 1)
            def _():
                fire(1, eoff, 0, gsem0)

        @pl.when(rn % 2 == 1)
        def _():
            @pl.when(sbn == 0)
            def _():
                fire(0, eoff, 1, gsem1)

            @pl.when(sbn == 1)
            def _():
                fire(1, eoff, 1, gsem1)

    def wait_roi(r):
        sb = (r // 8) % 2
        eoff = pl.multiple_of(NEP * (r % 8), 16)

        @pl.when(r % 2 == 0)
        def _():
            @pl.when(sb == 0)
            def _():
                wait(0, eoff, 0, gsem0)

            @pl.when(sb == 1)
            def _():
                wait(1, eoff, 0, gsem0)

        @pl.when(r % 2 == 1)
        def _():
            @pl.when(sb == 0)
            def _():
                wait(0, eoff, 1, gsem1)

            @pl.when(sb == 1)
            def _():
                wait(1, eoff, 1, gsem1)

    def compute(r):
        sb = (r // 8) % 2
        rb = r % 2
        wbase = NEP * (r % 8)

        def do_bin(k, e, wvec, wslot):
            w0 = wvec[wslot]
            w1 = wvec[wslot + 1]
            w2 = wvec[wslot + 2]
            w3 = wvec[wslot + 3]
            for g in range(C // 16):
                val = (
                    w0 * rows_v[rb, e, pl.ds(16 * g, 16)]
                    + w1 * rows_v[rb, e + 1, pl.ds(16 * g, 16)]
                    + w2 * rows_v[rb, e + 2, pl.ds(16 * g, 16)]
                    + w3 * rows_v[rb, e + 3, pl.ds(16 * g, 16)]
                )
                plsc.store_scatter(acc_v, (lane49 + (16 * g * NB + k),), val)

        def qbody(q, carry):
            wvec = sbw[pl.ds(pl.multiple_of(sb * 8 * NEP + wbase + 16 * q, 16), 16)]
            for bb in range(4):
                do_bin(4 * q + bb, 16 * q + 4 * bb, wvec, 4 * bb)
            return carry

        lax.fori_loop(0, 12, qbody, 0)
        wvec48 = sbw[pl.ds(pl.multiple_of(sb * 8 * NEP + wbase + 192, 16), 16)]
        do_bin(jnp.int32(48), 192, wvec48, 0)

    stage(0, 0)
    fire_roi(jnp.int32(0))

    def body(r, carry):
        @pl.when(r % 8 == 0)
        def _():
            blk = r // 8

            @pl.when(blk + 1 < NBLK)
            def _():
                @pl.when(blk % 2 == 0)
                def _():
                    stage(blk + 1, 1)

                @pl.when(blk % 2 == 1)
                def _():
                    stage(blk + 1, 0)

        @pl.when(r + 1 < RPW)
        def _():
            fire_roi(r + 1)

        wait_roi(r)

        @pl.when(r < count)
        def _():
            compute(r)
            pltpu.sync_copy(
                acc_v, out_hbm.at[pl.ds((rstart + r) * OUTR, OUTR)]
            )

        return carry

    lax.fori_loop(0, RPW, body, 0)


def kernel(feat0, feat1, feat2, feat3, rois):
    rois_p = jnp.pad(rois, ((0, RP - R), (0, 0)))
    idx_all, w_all = _tc_indices(rois_p)
    table = jnp.concatenate(
        [f.reshape(C, -1).T for f in (feat0, feat1, feat2, feat3)], axis=0
    )
    pooled = _sc_gather_fn()(idx_all.reshape(-1), w_all.reshape(-1), table)
    return pooled.reshape(R, C, PH, PW)


# speedup vs baseline: 1.4840x; 1.4840x over previous
import functools

import jax
import jax.numpy as jnp
from jax import lax
from jax.experimental import pallas as pl
from jax.experimental.pallas import tpu as pltpu
from jax.experimental.pallas import tpu_sc as plsc

PH, PW = 7, 7
R = 5000
RP = 5024
NB = PH * PW
NE = NB * 4
NEP = 208
C = 256
RB = 1256
OUTR = C * NB

LEVEL_W = (256, 128, 64, 32)
LEVEL_BASE = (0, 65536, 81920, 86016)

NW = 32
RPW = 160
NBLK = RPW // 8


def _tc_index_kernel(rois_ref, idx_ref, w_ref):
    rois = rois_ref[...]
    x1 = rois[:, 1:2]
    y1 = rois[:, 2:3]
    x2 = rois[:, 3:4]
    y2 = rois[:, 4:5]
    bw = x2 - x1 + 1.0
    bh = y2 - y1 + 1.0
    fid = jnp.clip(
        jnp.floor(2.0 + jnp.log2(jnp.sqrt(bw * bh) / 224.0 + 1e-6)), 0.0, 3.0
    ).astype(jnp.int32)
    scale = jnp.where(
        fid == 0, 0.25, jnp.where(fid == 1, 0.125, jnp.where(fid == 2, 0.0625, 0.03125))
    ).astype(jnp.float32)
    wl = jnp.where(fid == 0, LEVEL_W[0],
                   jnp.where(fid == 1, LEVEL_W[1],
                             jnp.where(fid == 2, LEVEL_W[2], LEVEL_W[3])))
    basel = jnp.where(fid == 0, LEVEL_BASE[0],
                      jnp.where(fid == 1, LEVEL_BASE[1],
                                jnp.where(fid == 2, LEVEL_BASE[2], LEVEL_BASE[3])))

    lane = lax.broadcasted_iota(jnp.int32, (RB, NEP), 1)
    live = lane < NE
    k = lane // 4
    corner = lane - 4 * k
    bi = k // PW
    bj = k - PW * bi
    dy = corner // 2
    dx = corner - 2 * dy

    x1s = x1 * scale
    y1s = y1 * scale
    roi_w = jnp.maximum(x2 * scale - x1s, 1.0)
    roi_h = jnp.maximum(y2 * scale - y1s, 1.0)
    bin_w = roi_w / PW
    bin_h = roi_h / PH
    px = x1s + (bj.astype(jnp.float32) + 0.5) * bin_w
    py = y1s + (bi.astype(jnp.float32) + 0.5) * bin_h
    x0f = jnp.floor(px)
    y0f = jnp.floor(py)
    lx = px - x0f
    ly = py - y0f
    hi = wl - 1
    x0 = jnp.clip(x0f.astype(jnp.int32), 0, hi)
    x1i = jnp.clip(x0 + 1, 0, hi)
    y0 = jnp.clip(y0f.astype(jnp.int32), 0, hi)
    y1i = jnp.clip(y0 + 1, 0, hi)
    ys = jnp.where(dy == 0, y0, y1i)
    xs = jnp.where(dx == 0, x0, x1i)
    wy = jnp.where(dy == 0, 1.0 - ly, ly)
    wx = jnp.where(dx == 0, 1.0 - lx, lx)
    idx_ref[...] = jnp.where(live, basel + ys * wl + xs, 0)
    w_ref[...] = jnp.where(live, wy * wx, 0.0)


def _tc_indices(rois, interpret=False):
    return pl.pallas_call(
        _tc_index_kernel,
        grid=(RP // RB,),
        in_specs=[pl.BlockSpec((RB, 5), lambda i: (i, 0))],
        out_specs=[
            pl.BlockSpec((RB, NEP), lambda i: (i, 0)),
            pl.BlockSpec((RB, NEP), lambda i: (i, 0)),
        ],
        out_shape=[
            jax.ShapeDtypeStruct((RP, NEP), jnp.int32),
            jax.ShapeDtypeStruct((RP, NEP), jnp.float32),
        ],
        interpret=interpret,
    )(rois)


@functools.cache
def _sc_gather_fn():
    mesh = plsc.VectorSubcoreMesh(
        core_axis_name="c", subcore_axis_name="s", num_cores=2, num_subcores=16
    )
    return functools.partial(
        pl.kernel,
        out_type=jax.ShapeDtypeStruct((R * OUTR,), jnp.float32),
        mesh=mesh,
        scratch_types=[
            pltpu.VMEM((2 * 8 * NEP,), jnp.int32),
            pltpu.VMEM((2 * 8 * NEP,), jnp.float32),
            pltpu.VMEM((2, NEP, C), jnp.float32),
            pltpu.VMEM((OUTR,), jnp.float32),
            pltpu.SemaphoreType.DMA,
            pltpu.SemaphoreType.DMA,
        ],
    )(_sc_gather_body)


def _sc_gather_body(idx_hbm, w_hbm, table_hbm, out_hbm, sbidx, sbw, rows_v, acc_v,
                    gsem0, gsem1):
    wid = lax.axis_index("s") * 2 + lax.axis_index("c")
    rstart = 156 * wid + jnp.minimum(wid, 8)
    count = jnp.where(wid < 8, 157, 156)

    def stage(blk, sb):
        off = (rstart + 8 * blk) * NEP
        pltpu.sync_copy(idx_hbm.at[pl.ds(off, 8 * NEP)],
                        sbidx.at[pl.ds(sb * 8 * NEP, 8 * NEP)])
        pltpu.sync_copy(w_hbm.at[pl.ds(off, 8 * NEP)],
                        sbw.at[pl.ds(sb * 8 * NEP, 8 * NEP)])

    def fire(sb, eoff, rb, sem):
        so = sb * 8 * NEP + eoff
        pltpu.async_copy(
            table_hbm.at[sbidx.at[pl.ds(so, 128)]],
            rows_v.at[rb, pl.ds(0, 128)], sem)
        pltpu.async_copy(
            table_hbm.at[sbidx.at[pl.ds(so + 128, 80)]],
            rows_v.at[rb, pl.ds(128, 80)], sem)

    def wait(sb, eoff, rb, sem):
        so = sb * 8 * NEP + eoff
        pltpu.make_async_copy(
            table_hbm.at[sbidx.at[pl.ds(so, 128)]],
            rows_v.at[rb, pl.ds(0, 128)], sem).wait()
        pltpu.make_async_copy(
            table_hbm.at[sbidx.at[pl.ds(so + 128, 80)]],
            rows_v.at[rb, pl.ds(128, 80)], sem).wait()

    def fire_roi(rn):
        sbn = (rn // 8) % 2
        eoff = pl.multiple_of(NEP * (rn % 8), 16)

        @pl.when(rn % 2 == 0)
        def _():
            @pl.when(sbn == 0)
            def _():
                fire(0, eoff, 0, gsem0)

            @pl.when(sbn == 1)
            def _():
                fire(1, eoff, 0, gsem0)

        @pl.when(rn % 2 == 1)
        def _():
            @pl.when(sbn == 0)
            def _():
                fire(0, eoff, 1, gsem1)

            @pl.when(sbn == 1)
            def _():
                fire(1, eoff, 1, gsem1)

    def wait_roi(r):
        sb = (r // 8) % 2
        eoff = pl.multiple_of(NEP * (r % 8), 16)

        @pl.when(r % 2 == 0)
        def _():
            @pl.when(sb == 0)
            def _():
                wait(0, eoff, 0, gsem0)

            @pl.when(sb == 1)
            def _():
                wait(1, eoff, 0, gsem0)

        @pl.when(r % 2 == 1)
        def _():
            @pl.when(sb == 0)
            def _():
                wait(0, eoff, 1, gsem1)

            @pl.when(sb == 1)
            def _():
                wait(1, eoff, 1, gsem1)

    def compute(r):
        sb = (r // 8) % 2
        rb = r % 2
        wbase = NEP * (r % 8)

        def do_bin(k, e, wvec, wslot):
            w0 = wvec[wslot]
            w1 = wvec[wslot + 1]
            w2 = wvec[wslot + 2]
            w3 = wvec[wslot + 3]
            for g in range(C // 16):
                val = (
                    w0 * rows_v[rb, e, pl.ds(16 * g, 16)]
                    + w1 * rows_v[rb, e + 1, pl.ds(16 * g, 16)]
                    + w2 * rows_v[rb, e + 2, pl.ds(16 * g, 16)]
                    + w3 * rows_v[rb, e + 3, pl.ds(16 * g, 16)]
                )
                acc_v[pl.ds(pl.multiple_of(k * C + 16 * g, 16), 16)] = val

        def qbody(q, carry):
            wvec = sbw[pl.ds(pl.multiple_of(sb * 8 * NEP + wbase + 16 * q, 16), 16)]
            for bb in range(4):
                do_bin(4 * q + bb, 16 * q + 4 * bb, wvec, 4 * bb)
            return carry

        lax.fori_loop(0, 12, qbody, 0)
        wvec48 = sbw[pl.ds(pl.multiple_of(sb * 8 * NEP + wbase + 192, 16), 16)]
        do_bin(jnp.int32(48), 192, wvec48, 0)

    stage(0, 0)
    fire_roi(jnp.int32(0))

    def body(r, carry):
        @pl.when(r % 8 == 0)
        def _():
            blk = r // 8

            @pl.when(blk + 1 < NBLK)
            def _():
                @pl.when(blk % 2 == 0)
                def _():
                    stage(blk + 1, 1)

                @pl.when(blk % 2 == 1)
                def _():
                    stage(blk + 1, 0)

        @pl.when(r + 1 < RPW)
        def _():
            fire_roi(r + 1)

        wait_roi(r)

        @pl.when(r < count)
        def _():
            compute(r)
            pltpu.sync_copy(
                acc_v, out_hbm.at[pl.ds((rstart + r) * OUTR, OUTR)]
            )

        return carry

    lax.fori_loop(0, RPW, body, 0)


def kernel(feat0, feat1, feat2, feat3, rois):
    rois_p = jnp.pad(rois, ((0, RP - R), (0, 0)))
    idx_all, w_all = _tc_indices(rois_p)
    table = jnp.concatenate(
        [f.reshape(C, -1).T for f in (feat0, feat1, feat2, feat3)], axis=0
    )
    pooled = _sc_gather_fn()(idx_all.reshape(-1), w_all.reshape(-1), table)
    return jnp.transpose(pooled.reshape(R, PH, PW, C), (0, 3, 1, 2))


# speedup vs baseline: 2.1014x; 1.4160x over previous
import functools

import jax
import jax.numpy as jnp
from jax import lax
from jax.experimental import pallas as pl
from jax.experimental.pallas import tpu as pltpu
from jax.experimental.pallas import tpu_sc as plsc

PH, PW = 7, 7
R = 5000
RP = 5024
NB = PH * PW
NE = NB * 4
NEP = 208
C = 256
RB = 1256
OUTR = C * NB

LEVEL_W = (256, 128, 64, 32)
LEVEL_BASE = (0, 65536, 81920, 86016)

NW = 32
RPW = 160
NBLK = RPW // 8


def _tc_index_kernel(rois_ref, idx_ref, w_ref):
    rois = rois_ref[...]
    x1 = rois[:, 1:2]
    y1 = rois[:, 2:3]
    x2 = rois[:, 3:4]
    y2 = rois[:, 4:5]
    bw = x2 - x1 + 1.0
    bh = y2 - y1 + 1.0
    fid = jnp.clip(
        jnp.floor(2.0 + jnp.log2(jnp.sqrt(bw * bh) / 224.0 + 1e-6)), 0.0, 3.0
    ).astype(jnp.int32)
    scale = jnp.where(
        fid == 0, 0.25, jnp.where(fid == 1, 0.125, jnp.where(fid == 2, 0.0625, 0.03125))
    ).astype(jnp.float32)
    wl = jnp.where(fid == 0, LEVEL_W[0],
                   jnp.where(fid == 1, LEVEL_W[1],
                             jnp.where(fid == 2, LEVEL_W[2], LEVEL_W[3])))
    basel = jnp.where(fid == 0, LEVEL_BASE[0],
                      jnp.where(fid == 1, LEVEL_BASE[1],
                                jnp.where(fid == 2, LEVEL_BASE[2], LEVEL_BASE[3])))

    lane = lax.broadcasted_iota(jnp.int32, (RB, NEP), 1)
    live = lane < NE
    k = lane // 4
    corner = lane - 4 * k
    bi = k // PW
    bj = k - PW * bi
    dy = corner // 2
    dx = corner - 2 * dy

    x1s = x1 * scale
    y1s = y1 * scale
    roi_w = jnp.maximum(x2 * scale - x1s, 1.0)
    roi_h = jnp.maximum(y2 * scale - y1s, 1.0)
    bin_w = roi_w / PW
    bin_h = roi_h / PH
    px = x1s + (bj.astype(jnp.float32) + 0.5) * bin_w
    py = y1s + (bi.astype(jnp.float32) + 0.5) * bin_h
    x0f = jnp.floor(px)
    y0f = jnp.floor(py)
    lx = px - x0f
    ly = py - y0f
    hi = wl - 1
    x0 = jnp.clip(x0f.astype(jnp.int32), 0, hi)
    x1i = jnp.clip(x0 + 1, 0, hi)
    y0 = jnp.clip(y0f.astype(jnp.int32), 0, hi)
    y1i = jnp.clip(y0 + 1, 0, hi)
    ys = jnp.where(dy == 0, y0, y1i)
    xs = jnp.where(dx == 0, x0, x1i)
    wy = jnp.where(dy == 0, 1.0 - ly, ly)
    wx = jnp.where(dx == 0, 1.0 - lx, lx)
    idx_ref[...] = jnp.where(live, basel + ys * wl + xs, 0)
    w_ref[...] = jnp.where(live, wy * wx, 0.0)


def _tc_indices(rois, interpret=False):
    return pl.pallas_call(
        _tc_index_kernel,
        grid=(RP // RB,),
        in_specs=[pl.BlockSpec((RB, 5), lambda i: (i, 0))],
        out_specs=[
            pl.BlockSpec((RB, NEP), lambda i: (i, 0)),
            pl.BlockSpec((RB, NEP), lambda i: (i, 0)),
        ],
        out_shape=[
            jax.ShapeDtypeStruct((RP, NEP), jnp.int32),
            jax.ShapeDtypeStruct((RP, NEP), jnp.float32),
        ],
        interpret=interpret,
    )(rois)


@functools.cache
def _sc_gather_fn():
    mesh = plsc.VectorSubcoreMesh(
        core_axis_name="c", subcore_axis_name="s", num_cores=2, num_subcores=16
    )
    return functools.partial(
        pl.kernel,
        out_type=jax.ShapeDtypeStruct((NB, R, C), jnp.float32),
        mesh=mesh,
        scratch_types=[
            pltpu.VMEM((2 * 8 * NEP,), jnp.int32),
            pltpu.VMEM((2 * 8 * NEP,), jnp.float32),
            pltpu.VMEM((2, NEP, C), jnp.float32),
            pltpu.VMEM((NB, C), jnp.float32),
            pltpu.SemaphoreType.DMA,
            pltpu.SemaphoreType.DMA,
        ],
    )(_sc_gather_body)


def _sc_gather_body(idx_hbm, w_hbm, table_hbm, out_hbm, sbidx, sbw, rows_v, acc_v,
                    gsem0, gsem1):
    wid = lax.axis_index("s") * 2 + lax.axis_index("c")
    rstart = jnp.where(wid < 17, 160 * wid, 2720 + 152 * (wid - 17))
    count = jnp.where(wid < 17, 160, 152)

    def stage(blk, sb):
        off = (rstart + 8 * blk) * NEP
        pltpu.sync_copy(idx_hbm.at[pl.ds(off, 8 * NEP)],
                        sbidx.at[pl.ds(sb * 8 * NEP, 8 * NEP)])
        pltpu.sync_copy(w_hbm.at[pl.ds(off, 8 * NEP)],
                        sbw.at[pl.ds(sb * 8 * NEP, 8 * NEP)])

    def fire(sb, eoff, rb, sem):
        so = sb * 8 * NEP + eoff
        pltpu.async_copy(
            table_hbm.at[sbidx.at[pl.ds(so, 128)]],
            rows_v.at[rb, pl.ds(0, 128)], sem)
        pltpu.async_copy(
            table_hbm.at[sbidx.at[pl.ds(so + 128, 80)]],
            rows_v.at[rb, pl.ds(128, 80)], sem)

    def wait(sb, eoff, rb, sem):
        so = sb * 8 * NEP + eoff
        pltpu.make_async_copy(
            table_hbm.at[sbidx.at[pl.ds(so, 128)]],
            rows_v.at[rb, pl.ds(0, 128)], sem).wait()
        pltpu.make_async_copy(
            table_hbm.at[sbidx.at[pl.ds(so + 128, 80)]],
            rows_v.at[rb, pl.ds(128, 80)], sem).wait()

    def fire_roi(rn):
        sbn = (rn // 8) % 2
        eoff = pl.multiple_of(NEP * (rn % 8), 16)

        @pl.when(rn % 2 == 0)
        def _():
            @pl.when(sbn == 0)
            def _():
                fire(0, eoff, 0, gsem0)

            @pl.when(sbn == 1)
            def _():
                fire(1, eoff, 0, gsem0)

        @pl.when(rn % 2 == 1)
        def _():
            @pl.when(sbn == 0)
            def _():
                fire(0, eoff, 1, gsem1)

            @pl.when(sbn == 1)
            def _():
                fire(1, eoff, 1, gsem1)

    def wait_roi(r):
        sb = (r // 8) % 2
        eoff = pl.multiple_of(NEP * (r % 8), 16)

        @pl.when(r % 2 == 0)
        def _():
            @pl.when(sb == 0)
            def _():
                wait(0, eoff, 0, gsem0)

            @pl.when(sb == 1)
            def _():
                wait(1, eoff, 0, gsem0)

        @pl.when(r % 2 == 1)
        def _():
            @pl.when(sb == 0)
            def _():
                wait(0, eoff, 1, gsem1)

            @pl.when(sb == 1)
            def _():
                wait(1, eoff, 1, gsem1)

    def compute(r):
        sb = (r // 8) % 2
        rb = r % 2
        wbase = NEP * (r % 8)

        def do_bin(k, e, wvec, wslot):
            w0 = wvec[wslot]
            w1 = wvec[wslot + 1]
            w2 = wvec[wslot + 2]
            w3 = wvec[wslot + 3]
            for g in range(C // 16):
                val = (
                    w0 * rows_v[rb, e, pl.ds(16 * g, 16)]
                    + w1 * rows_v[rb, e + 1, pl.ds(16 * g, 16)]
                    + w2 * rows_v[rb, e + 2, pl.ds(16 * g, 16)]
                    + w3 * rows_v[rb, e + 3, pl.ds(16 * g, 16)]
                )
                acc_v[k, pl.ds(16 * g, 16)] = val

        def qbody(q, carry):
            wvec = sbw[pl.ds(pl.multiple_of(sb * 8 * NEP + wbase + 16 * q, 16), 16)]
            for bb in range(4):
                do_bin(4 * q + bb, 16 * q + 4 * bb, wvec, 4 * bb)
            return carry

        lax.fori_loop(0, 12, qbody, 0)
        wvec48 = sbw[pl.ds(pl.multiple_of(sb * 8 * NEP + wbase + 192, 16), 16)]
        do_bin(jnp.int32(48), 192, wvec48, 0)

    stage(0, 0)
    fire_roi(jnp.int32(0))

    def body(r, carry):
        @pl.when(r % 8 == 0)
        def _():
            blk = r // 8

            @pl.when(blk + 1 < NBLK)
            def _():
                @pl.when(blk % 2 == 0)
                def _():
                    stage(blk + 1, 1)

                @pl.when(blk % 2 == 1)
                def _():
                    stage(blk + 1, 0)

        @pl.when(r + 1 < RPW)
        def _():
            fire_roi(r + 1)

        wait_roi(r)

        @pl.when(r < count)
        def _():
            compute(r)
            pltpu.sync_copy(acc_v, out_hbm.at[:, rstart + r, :])

        return carry

    lax.fori_loop(0, RPW, body, 0)


def kernel(feat0, feat1, feat2, feat3, rois):
    rois_p = jnp.pad(rois, ((0, RP - R), (0, 0)))
    idx_all, w_all = _tc_indices(rois_p)
    table = jnp.concatenate(
        [f.reshape(C, -1).T for f in (feat0, feat1, feat2, feat3)], axis=0
    )
    pooled = _sc_gather_fn()(idx_all.reshape(-1), w_all.reshape(-1), table)
    return jnp.transpose(pooled.reshape(PH, PW, R, C), (2, 3, 0, 1))


# speedup vs baseline: 3.6073x; 1.7166x over previous
import functools

import jax
import jax.numpy as jnp
from jax import lax
from jax.experimental import pallas as pl
from jax.experimental.pallas import tpu as pltpu
from jax.experimental.pallas import tpu_sc as plsc

PH, PW = 7, 7
R = 5000
NB = PH * PW
NE = NB * 4
C = 256
RB = 1000

LEVEL_W = (256, 128, 64, 32)
LEVEL_BASE = (0, 65536, 81920, 86016)

NW = 32
CH4 = 128
M1 = R * NB
M4 = 4 * M1
PER_W4 = -(-M4 // (NW * CH4)) * CH4
CHUNKS = PER_W4 // CH4
M4_PAD = PER_W4 * NW


def _tc_index_kernel(rois_ref, idx_ref, w_ref):
    rois = rois_ref[...]
    x1 = rois[:, 1:2]
    y1 = rois[:, 2:3]
    x2 = rois[:, 3:4]
    y2 = rois[:, 4:5]
    bw = x2 - x1 + 1.0
    bh = y2 - y1 + 1.0
    fid = jnp.clip(
        jnp.floor(2.0 + jnp.log2(jnp.sqrt(bw * bh) / 224.0 + 1e-6)), 0.0, 3.0
    ).astype(jnp.int32)
    scale = jnp.where(
        fid == 0, 0.25, jnp.where(fid == 1, 0.125, jnp.where(fid == 2, 0.0625, 0.03125))
    ).astype(jnp.float32)
    wl = jnp.where(fid == 0, LEVEL_W[0],
                   jnp.where(fid == 1, LEVEL_W[1],
                             jnp.where(fid == 2, LEVEL_W[2], LEVEL_W[3])))
    basel = jnp.where(fid == 0, LEVEL_BASE[0],
                      jnp.where(fid == 1, LEVEL_BASE[1],
                                jnp.where(fid == 2, LEVEL_BASE[2], LEVEL_BASE[3])))

    lane = lax.broadcasted_iota(jnp.int32, (RB, NE), 1)
    k = lane // 4
    corner = lane - 4 * k
    bi = k // PW
    bj = k - PW * bi
    dy = corner // 2
    dx = corner - 2 * dy

    x1s = x1 * scale
    y1s = y1 * scale
    roi_w = jnp.maximum(x2 * scale - x1s, 1.0)
    roi_h = jnp.maximum(y2 * scale - y1s, 1.0)
    bin_w = roi_w / PW
    bin_h = roi_h / PH
    px = x1s + (bj.astype(jnp.float32) + 0.5) * bin_w
    py = y1s + (bi.astype(jnp.float32) + 0.5) * bin_h
    x0f = jnp.floor(px)
    y0f = jnp.floor(py)
    lx = px - x0f
    ly = py - y0f
    hi = wl - 1
    x0 = jnp.clip(x0f.astype(jnp.int32), 0, hi)
    x1i = jnp.clip(x0 + 1, 0, hi)
    y0 = jnp.clip(y0f.astype(jnp.int32), 0, hi)
    y1i = jnp.clip(y0 + 1, 0, hi)
    ys = jnp.where(dy == 0, y0, y1i)
    xs = jnp.where(dx == 0, x0, x1i)
    wy = jnp.where(dy == 0, 1.0 - ly, ly)
    wx = jnp.where(dx == 0, 1.0 - lx, lx)
    idx_ref[...] = basel + ys * wl + xs
    w_ref[...] = wy * wx


def _tc_indices(rois, interpret=False):
    return pl.pallas_call(
        _tc_index_kernel,
        grid=(R // RB,),
        in_specs=[pl.BlockSpec((RB, 5), lambda i: (i, 0))],
        out_specs=[
            pl.BlockSpec((RB, NE), lambda i: (i, 0)),
            pl.BlockSpec((RB, NE), lambda i: (i, 0)),
        ],
        out_shape=[
            jax.ShapeDtypeStruct((R, NE), jnp.int32),
            jax.ShapeDtypeStruct((R, NE), jnp.float32),
        ],
        interpret=interpret,
    )(rois)


@functools.cache
def _sc_gather_fn():
    mesh = plsc.VectorSubcoreMesh(
        core_axis_name="c", subcore_axis_name="s", num_cores=2, num_subcores=16
    )
    return functools.partial(
        pl.kernel,
        out_type=jax.ShapeDtypeStruct((M1, C), jnp.float32),
        mesh=mesh,
        scratch_types=[
            pltpu.VMEM((2, CH4), jnp.int32),
            pltpu.VMEM((2, CH4), jnp.float32),
            pltpu.VMEM((2, CH4, C), jnp.float32),
            pltpu.VMEM((CH4 // 4, C), jnp.float32),
            pltpu.SemaphoreType.DMA,
            pltpu.SemaphoreType.DMA,
        ],
    )(_sc_gather_body)


def _sc_gather_body(idx_hbm, w_hbm, table_hbm, out_hbm, idx_v, w_v, rows_v, acc_v,
                    gsem0, gsem1):
    wid = lax.axis_index("s") * 2 + lax.axis_index("c")
    base4 = wid * PER_W4
    baser = wid * (PER_W4 // 4)
    sems = (gsem0, gsem1)

    def stage(tt, buf):
        off = base4 + tt * CH4
        pltpu.sync_copy(idx_hbm.at[pl.ds(off, CH4)], idx_v.at[buf])
        pltpu.sync_copy(w_hbm.at[pl.ds(off, CH4)], w_v.at[buf])
        pltpu.async_copy(table_hbm.at[idx_v.at[buf]], rows_v.at[buf], sems[buf])

    def wait(buf):
        pltpu.make_async_copy(
            table_hbm.at[idx_v.at[buf]], rows_v.at[buf], sems[buf]
        ).wait()

    def compute_and_store(tt, buf):
        def body(q, carry):
            wvec = w_v[buf, pl.ds(16 * q, 16)]
            for bb in range(4):
                b = 4 * q + bb
                r = 4 * b
                w0 = wvec[4 * bb]
                w1 = wvec[4 * bb + 1]
                w2 = wvec[4 * bb + 2]
                w3 = wvec[4 * bb + 3]
                for g in range(C // 16):
                    sl = pl.ds(16 * g, 16)
                    acc_v[b, sl] = (
                        w0 * rows_v[buf, r, sl]
                        + w1 * rows_v[buf, r + 1, sl]
                    ) + (
                        w2 * rows_v[buf, r + 2, sl]
                        + w3 * rows_v[buf, r + 3, sl]
                    )
            return carry

        lax.fori_loop(0, CH4 // 16, body, 0)
        base = baser + tt * (CH4 // 4)

        @pl.when(base + CH4 // 4 <= M1)
        def _():
            pltpu.sync_copy(acc_v, out_hbm.at[pl.ds(base, CH4 // 4)])

        @pl.when(base == M1 - 8)
        def _():
            pltpu.sync_copy(acc_v.at[pl.ds(0, 8)], out_hbm.at[pl.ds(base, 8)])

    stage(0, 0)

    def outer(t2, carry):
        tt0 = 2 * t2

        @pl.when(tt0 + 1 < CHUNKS)
        def _():
            stage(tt0 + 1, 1)

        wait(0)
        compute_and_store(tt0, 0)

        @pl.when(tt0 + 2 < CHUNKS)
        def _():
            stage(tt0 + 2, 0)

        @pl.when(tt0 + 1 < CHUNKS)
        def _():
            wait(1)
            compute_and_store(tt0 + 1, 1)

        return carry

    lax.fori_loop(0, (CHUNKS + 1) // 2, outer, 0)


def kernel(feat0, feat1, feat2, feat3, rois):
    idx_all, w_all = _tc_indices(rois)
    pad = M4_PAD - M4
    idx_p = jnp.transpose(idx_all.reshape(R, NB, 4), (1, 0, 2)).reshape(-1)
    w_p = jnp.transpose(w_all.reshape(R, NB, 4), (1, 0, 2)).reshape(-1)
    idx_flat = jnp.concatenate([idx_p, jnp.zeros((pad,), jnp.int32)])
    w_flat = jnp.concatenate([w_p, jnp.zeros((pad,), jnp.float32)])
    table = jnp.concatenate(
        [f.reshape(C, -1).T for f in (feat0, feat1, feat2, feat3)], axis=0
    )
    pooled = _sc_gather_fn()(idx_flat, w_flat, table)
    return jnp.transpose(pooled.reshape(PH, PW, R, C), (2, 3, 0, 1))


# speedup vs baseline: 5.9214x; 1.6415x over previous
import functools

import jax
import jax.numpy as jnp
from jax import lax
from jax.experimental import pallas as pl
from jax.experimental.pallas import tpu as pltpu
from jax.experimental.pallas import tpu_sc as plsc

PH, PW = 7, 7
R = 5000
NB = PH * PW
NE = NB * 4
C = 256
RB = 1000

LEVEL_W = (256, 128, 64, 32)
LEVEL_BASE = (0, 65536, 81920, 86016)

NW = 32
CH4 = 128
M1 = R * NB
M4 = 4 * M1
PER_W4 = -(-M4 // (NW * CH4)) * CH4
CHUNKS = PER_W4 // CH4
M4_PAD = PER_W4 * NW


def _tc_index_kernel(rois_ref, idx_ref, w_ref):
    rois = rois_ref[...]
    x1 = rois[:, 1:2]
    y1 = rois[:, 2:3]
    x2 = rois[:, 3:4]
    y2 = rois[:, 4:5]
    bw = x2 - x1 + 1.0
    bh = y2 - y1 + 1.0
    fid = jnp.clip(
        jnp.floor(2.0 + jnp.log2(jnp.sqrt(bw * bh) / 224.0 + 1e-6)), 0.0, 3.0
    ).astype(jnp.int32)
    scale = jnp.where(
        fid == 0, 0.25, jnp.where(fid == 1, 0.125, jnp.where(fid == 2, 0.0625, 0.03125))
    ).astype(jnp.float32)
    wl = jnp.where(fid == 0, LEVEL_W[0],
                   jnp.where(fid == 1, LEVEL_W[1],
                             jnp.where(fid == 2, LEVEL_W[2], LEVEL_W[3])))
    basel = jnp.where(fid == 0, LEVEL_BASE[0],
                      jnp.where(fid == 1, LEVEL_BASE[1],
                                jnp.where(fid == 2, LEVEL_BASE[2], LEVEL_BASE[3])))

    lane = lax.broadcasted_iota(jnp.int32, (RB, NE), 1)
    k = lane // 4
    corner = lane - 4 * k
    bi = k // PW
    bj = k - PW * bi
    dy = corner // 2
    dx = corner - 2 * dy

    x1s = x1 * scale
    y1s = y1 * scale
    roi_w = jnp.maximum(x2 * scale - x1s, 1.0)
    roi_h = jnp.maximum(y2 * scale - y1s, 1.0)
    bin_w = roi_w / PW
    bin_h = roi_h / PH
    px = x1s + (bj.astype(jnp.float32) + 0.5) * bin_w
    py = y1s + (bi.astype(jnp.float32) + 0.5) * bin_h
    x0f = jnp.floor(px)
    y0f = jnp.floor(py)
    lx = px - x0f
    ly = py - y0f
    hi = wl - 1
    x0 = jnp.clip(x0f.astype(jnp.int32), 0, hi)
    x1i = jnp.clip(x0 + 1, 0, hi)
    y0 = jnp.clip(y0f.astype(jnp.int32), 0, hi)
    y1i = jnp.clip(y0 + 1, 0, hi)
    ys = jnp.where(dy == 0, y0, y1i)
    xs = jnp.where(dx == 0, x0, x1i)
    wy = jnp.where(dy == 0, 1.0 - ly, ly)
    wx = jnp.where(dx == 0, 1.0 - lx, lx)
    idx_ref[...] = basel + ys * wl + xs
    w_ref[...] = wy * wx


def _tc_indices(rois, interpret=False):
    return pl.pallas_call(
        _tc_index_kernel,
        grid=(R // RB,),
        in_specs=[pl.BlockSpec((RB, 5), lambda i: (i, 0))],
        out_specs=[
            pl.BlockSpec((RB, NE), lambda i: (i, 0)),
            pl.BlockSpec((RB, NE), lambda i: (i, 0)),
        ],
        out_shape=[
            jax.ShapeDtypeStruct((R, NE), jnp.int32),
            jax.ShapeDtypeStruct((R, NE), jnp.float32),
        ],
        interpret=interpret,
    )(rois)


@functools.cache
def _sc_gather_fn():
    mesh = plsc.VectorSubcoreMesh(
        core_axis_name="c", subcore_axis_name="s", num_cores=2, num_subcores=16
    )
    return functools.partial(
        pl.kernel,
        out_type=jax.ShapeDtypeStruct((M1, C), jnp.float32),
        mesh=mesh,
        scratch_types=[
            pltpu.VMEM((2, CH4), jnp.int32),
            pltpu.VMEM((2, CH4), jnp.float32),
            pltpu.VMEM((2, CH4, C), jnp.float32),
            pltpu.VMEM((CH4 // 4, C), jnp.float32),
            pltpu.SemaphoreType.DMA,
            pltpu.SemaphoreType.DMA,
        ],
    )(_sc_gather_body)


def _sc_gather_body(idx_hbm, w_hbm, table_hbm, out_hbm, idx_v, w_v, rows_v, acc_v,
                    gsem0, gsem1):
    wid = lax.axis_index("s") * 2 + lax.axis_index("c")
    base4 = wid * PER_W4
    baser = wid * (PER_W4 // 4)
    sems = (gsem0, gsem1)

    def stage(tt, buf):
        off = base4 + tt * CH4
        pltpu.sync_copy(idx_hbm.at[pl.ds(off, CH4)], idx_v.at[buf])
        pltpu.sync_copy(w_hbm.at[pl.ds(off, CH4)], w_v.at[buf])
        pltpu.async_copy(table_hbm.at[idx_v.at[buf]], rows_v.at[buf], sems[buf])

    def wait(buf):
        pltpu.make_async_copy(
            table_hbm.at[idx_v.at[buf]], rows_v.at[buf], sems[buf]
        ).wait()

    def compute_and_store(tt, buf):
        def body(q, carry):
            wvec = w_v[buf, pl.ds(16 * q, 16)]
            for bb in range(4):
                b = 4 * q + bb
                r = 4 * b
                w0 = wvec[4 * bb]
                w1 = wvec[4 * bb + 1]
                w2 = wvec[4 * bb + 2]
                w3 = wvec[4 * bb + 3]
                for g2 in range(4):
                    gs = [4 * g2 + i for i in range(4)]
                    loads = [
                        (
                            rows_v[buf, r, pl.ds(16 * g, 16)],
                            rows_v[buf, r + 1, pl.ds(16 * g, 16)],
                            rows_v[buf, r + 2, pl.ds(16 * g, 16)],
                            rows_v[buf, r + 3, pl.ds(16 * g, 16)],
                        )
                        for g in gs
                    ]
                    outs = [
                        (w0 * a + w1 * e) + (w2 * f + w3 * d)
                        for (a, e, f, d) in loads
                    ]
                    for g, v in zip(gs, outs):
                        acc_v[b, pl.ds(16 * g, 16)] = v
            return carry

        lax.fori_loop(0, CH4 // 16, body, 0)
        base = baser + tt * (CH4 // 4)

        @pl.when(base + CH4 // 4 <= M1)
        def _():
            pltpu.sync_copy(acc_v, out_hbm.at[pl.ds(base, CH4 // 4)])

        @pl.when(base == M1 - 8)
        def _():
            pltpu.sync_copy(acc_v.at[pl.ds(0, 8)], out_hbm.at[pl.ds(base, 8)])

    stage(0, 0)

    def outer(t2, carry):
        tt0 = 2 * t2

        @pl.when(tt0 + 1 < CHUNKS)
        def _():
            stage(tt0 + 1, 1)

        wait(0)
        compute_and_store(tt0, 0)

        @pl.when(tt0 + 2 < CHUNKS)
        def _():
            stage(tt0 + 2, 0)

        @pl.when(tt0 + 1 < CHUNKS)
        def _():
            wait(1)
            compute_and_store(tt0 + 1, 1)

        return carry

    lax.fori_loop(0, (CHUNKS + 1) // 2, outer, 0)


def kernel(feat0, feat1, feat2, feat3, rois):
    idx_all, w_all = _tc_indices(rois)
    pad = M4_PAD - M4
    idx_p = jnp.transpose(idx_all.reshape(R, NB, 4), (1, 0, 2)).reshape(-1)
    w_p = jnp.transpose(w_all.reshape(R, NB, 4), (1, 0, 2)).reshape(-1)
    idx_flat = jnp.concatenate([idx_p, jnp.zeros((pad,), jnp.int32)])
    w_flat = jnp.concatenate([w_p, jnp.zeros((pad,), jnp.float32)])
    table = jnp.concatenate(
        [f.reshape(C, -1).T for f in (feat0, feat1, feat2, feat3)], axis=0
    )
    pooled = _sc_gather_fn()(idx_flat, w_flat, table)
    return jnp.transpose(pooled.reshape(PH, PW, R, C), (2, 3, 0, 1))


# speedup vs baseline: 5.9240x; 1.0004x over previous
import functools

import jax
import jax.numpy as jnp
from jax import lax
from jax.experimental import pallas as pl
from jax.experimental.pallas import tpu as pltpu
from jax.experimental.pallas import tpu_sc as plsc

PH, PW = 7, 7
R = 5000
NB = PH * PW
NE = NB * 4
C = 256
RB = 1000

LEVEL_W = (256, 128, 64, 32)
LEVEL_BASE = (0, 65536, 81920, 86016)

NW = 32
CH4 = 128
M1 = R * NB
M4 = 4 * M1
PER_W4 = -(-M4 // (NW * CH4)) * CH4
CHUNKS = PER_W4 // CH4
M4_PAD = PER_W4 * NW
SBLK = 16


def _tc_index_kernel(rois_ref, idx_ref, w_ref):
    rois = rois_ref[...]
    x1 = rois[:, 1:2]
    y1 = rois[:, 2:3]
    x2 = rois[:, 3:4]
    y2 = rois[:, 4:5]
    bw = x2 - x1 + 1.0
    bh = y2 - y1 + 1.0
    fid = jnp.clip(
        jnp.floor(2.0 + jnp.log2(jnp.sqrt(bw * bh) / 224.0 + 1e-6)), 0.0, 3.0
    ).astype(jnp.int32)
    scale = jnp.where(
        fid == 0, 0.25, jnp.where(fid == 1, 0.125, jnp.where(fid == 2, 0.0625, 0.03125))
    ).astype(jnp.float32)
    wl = jnp.where(fid == 0, LEVEL_W[0],
                   jnp.where(fid == 1, LEVEL_W[1],
                             jnp.where(fid == 2, LEVEL_W[2], LEVEL_W[3])))
    basel = jnp.where(fid == 0, LEVEL_BASE[0],
                      jnp.where(fid == 1, LEVEL_BASE[1],
                                jnp.where(fid == 2, LEVEL_BASE[2], LEVEL_BASE[3])))

    lane = lax.broadcasted_iota(jnp.int32, (RB, NE), 1)
    k = lane // 4
    corner = lane - 4 * k
    bi = k // PW
    bj = k - PW * bi
    dy = corner // 2
    dx = corner - 2 * dy

    x1s = x1 * scale
    y1s = y1 * scale
    roi_w = jnp.maximum(x2 * scale - x1s, 1.0)
    roi_h = jnp.maximum(y2 * scale - y1s, 1.0)
    bin_w = roi_w / PW
    bin_h = roi_h / PH
    px = x1s + (bj.astype(jnp.float32) + 0.5) * bin_w
    py = y1s + (bi.astype(jnp.float32) + 0.5) * bin_h
    x0f = jnp.floor(px)
    y0f = jnp.floor(py)
    lx = px - x0f
    ly = py - y0f
    hi = wl - 1
    x0 = jnp.clip(x0f.astype(jnp.int32), 0, hi)
    x1i = jnp.clip(x0 + 1, 0, hi)
    y0 = jnp.clip(y0f.astype(jnp.int32), 0, hi)
    y1i = jnp.clip(y0 + 1, 0, hi)
    ys = jnp.where(dy == 0, y0, y1i)
    xs = jnp.where(dx == 0, x0, x1i)
    wy = jnp.where(dy == 0, 1.0 - ly, ly)
    wx = jnp.where(dx == 0, 1.0 - lx, lx)
    idx_ref[...] = basel + ys * wl + xs
    w_ref[...] = wy * wx


def _tc_indices(rois, interpret=False):
    return pl.pallas_call(
        _tc_index_kernel,
        grid=(R // RB,),
        in_specs=[pl.BlockSpec((RB, 5), lambda i: (i, 0))],
        out_specs=[
            pl.BlockSpec((RB, NE), lambda i: (i, 0)),
            pl.BlockSpec((RB, NE), lambda i: (i, 0)),
        ],
        out_shape=[
            jax.ShapeDtypeStruct((R, NE), jnp.int32),
            jax.ShapeDtypeStruct((R, NE), jnp.float32),
        ],
        interpret=interpret,
    )(rois)


@functools.cache
def _sc_gather_fn():
    mesh = plsc.VectorSubcoreMesh(
        core_axis_name="c", subcore_axis_name="s", num_cores=2, num_subcores=16
    )
    return functools.partial(
        pl.kernel,
        out_type=jax.ShapeDtypeStruct((M1, C), jnp.float32),
        mesh=mesh,
        scratch_types=[
            pltpu.VMEM((2 * SBLK * CH4,), jnp.int32),
            pltpu.VMEM((2 * SBLK * CH4,), jnp.float32),
            pltpu.VMEM((2, CH4, C), jnp.float32),
            pltpu.VMEM((2, CH4 // 4, C), jnp.float32),
            pltpu.SemaphoreType.DMA,
            pltpu.SemaphoreType.DMA,
            pltpu.SemaphoreType.DMA,
            pltpu.SemaphoreType.DMA,
        ],
    )(_sc_gather_body)


def _sc_gather_body(idx_hbm, w_hbm, table_hbm, out_hbm, sbidx, sbw, rows_v, acc_v,
                    gsem0, gsem1, osem0, osem1):
    wid = lax.axis_index("s") * 2 + lax.axis_index("c")
    base4 = wid * PER_W4
    baser = wid * (PER_W4 // 4)
    sems = (gsem0, gsem1)
    osems = (osem0, osem1)

    def stage_block(blk, sb):
        off = base4 + blk * (SBLK * CH4)
        pltpu.sync_copy(idx_hbm.at[pl.ds(off, SBLK * CH4)],
                        sbidx.at[pl.ds(sb * SBLK * CH4, SBLK * CH4)])
        pltpu.sync_copy(w_hbm.at[pl.ds(off, SBLK * CH4)],
                        sbw.at[pl.ds(sb * SBLK * CH4, SBLK * CH4)])

    def idx_slice(tt):
        so = (tt // SBLK) % 2 * (SBLK * CH4) + (tt % SBLK) * CH4
        return sbidx.at[pl.ds(so, CH4)]

    def stage(tt, buf):
        pltpu.async_copy(table_hbm.at[idx_slice(tt)], rows_v.at[buf], sems[buf])

    def wait(tt, buf):
        pltpu.make_async_copy(
            table_hbm.at[idx_slice(tt)], rows_v.at[buf], sems[buf]
        ).wait()

    def compute_and_store(tt, buf):
        wbase = (tt // SBLK) % 2 * (SBLK * CH4) + (tt % SBLK) * CH4

        def body(q, carry):
            wvec = sbw[pl.ds(pl.multiple_of(wbase + 16 * q, 16), 16)]
            for bb in range(4):
                b = 4 * q + bb
                r = 4 * b
                w0 = wvec[4 * bb]
                w1 = wvec[4 * bb + 1]
                w2 = wvec[4 * bb + 2]
                w3 = wvec[4 * bb + 3]
                for g2 in range(4):
                    gs = [4 * g2 + i for i in range(4)]
                    loads = [
                        (
                            rows_v[buf, r, pl.ds(16 * g, 16)],
                            rows_v[buf, r + 1, pl.ds(16 * g, 16)],
                            rows_v[buf, r + 2, pl.ds(16 * g, 16)],
                            rows_v[buf, r + 3, pl.ds(16 * g, 16)],
                        )
                        for g in gs
                    ]
                    outs = [
                        (w0 * a + w1 * e) + (w2 * f + w3 * d)
                        for (a, e, f, d) in loads
                    ]
                    for g, v in zip(gs, outs):
                        acc_v[buf, b, pl.ds(16 * g, 16)] = v
            return carry

        lax.fori_loop(0, CH4 // 16, body, 0)
        base = baser + tt * (CH4 // 4)

        @pl.when(base + CH4 // 4 <= M1)
        def _():
            pltpu.async_copy(
                acc_v.at[buf], out_hbm.at[pl.ds(base, CH4 // 4)], osems[buf])

        @pl.when(base == M1 - 8)
        def _():
            pltpu.async_copy(
                acc_v.at[buf, pl.ds(0, 8)], out_hbm.at[pl.ds(base, 8)],
                osems[buf])

    def wait_out(tt, buf):
        base = baser + tt * (CH4 // 4)

        @pl.when(base + CH4 // 4 <= M1)
        def _():
            pltpu.make_async_copy(
                acc_v.at[buf], out_hbm.at[pl.ds(base, CH4 // 4)], osems[buf]
            ).wait()

        @pl.when(base == M1 - 8)
        def _():
            pltpu.make_async_copy(
                acc_v.at[buf, pl.ds(0, 8)], out_hbm.at[pl.ds(base, 8)],
                osems[buf]
            ).wait()

    stage_block(0, 0)
    stage(0, 0)

    def outer(t2, carry):
        tt0 = 2 * t2

        @pl.when(tt0 % SBLK == 0)
        def _():
            blk = tt0 // SBLK

            @pl.when(blk + 1 < CHUNKS // SBLK)
            def _():
                @pl.when(blk % 2 == 0)
                def _():
                    stage_block(blk + 1, 1)

                @pl.when(blk % 2 == 1)
                def _():
                    stage_block(blk + 1, 0)

        @pl.when(tt0 + 1 < CHUNKS)
        def _():
            stage(tt0 + 1, 1)

        wait(tt0, 0)

        @pl.when(tt0 >= 2)
        def _():
            wait_out(tt0 - 2, 0)

        compute_and_store(tt0, 0)

        @pl.when(tt0 + 2 < CHUNKS)
        def _():
            stage(tt0 + 2, 0)

        @pl.when(tt0 + 1 < CHUNKS)
        def _():
            wait(tt0 + 1, 1)

            @pl.when(tt0 >= 1)
            def _():
                wait_out(tt0 - 1, 1)

            compute_and_store(tt0 + 1, 1)

        return carry

    lax.fori_loop(0, (CHUNKS + 1) // 2, outer, 0)
    wait_out(CHUNKS - 2, 0)
    wait_out(CHUNKS - 1, 1)


def kernel(feat0, feat1, feat2, feat3, rois):
    idx_all, w_all = _tc_indices(rois)
    pad = M4_PAD - M4
    idx_p = jnp.transpose(idx_all.reshape(R, NB, 4), (1, 0, 2)).reshape(-1)
    w_p = jnp.transpose(w_all.reshape(R, NB, 4), (1, 0, 2)).reshape(-1)
    idx_flat = jnp.concatenate([idx_p, jnp.zeros((pad,), jnp.int32)])
    w_flat = jnp.concatenate([w_p, jnp.zeros((pad,), jnp.float32)])
    table = jnp.concatenate(
        [f.reshape(C, -1).T for f in (feat0, feat1, feat2, feat3)], axis=0
    )
    pooled = _sc_gather_fn()(idx_flat, w_flat, table)
    return jnp.transpose(pooled.reshape(PH, PW, R, C), (2, 3, 0, 1))
